# baseline (speedup 1.0000x reference)
import functools

import jax
import jax.numpy as jnp
from jax import lax
from jax.experimental import pallas as pl
from jax.experimental.pallas import tpu as pltpu
from jax.experimental.pallas import tpu_sc as plsc

B = 16384
D = 32
IDX_CHUNK = 128
DRAIN_C = 8


@functools.lru_cache(maxsize=None)
def _make_gather(num_cores, num_subcores):
    NC, NS = num_cores, num_subcores
    NW = NC * NS
    b_per_w = B // NW
    n_chunks = b_per_w // IDX_CHUNK
    mesh = plsc.VectorSubcoreMesh(core_axis_name="c", subcore_axis_name="s")

    @functools.partial(
        pl.kernel,
        mesh=mesh,
        compiler_params=pltpu.CompilerParams(use_tc_tiling_on_sc=False),
        out_type=[
            jax.ShapeDtypeStruct((D, B), jnp.float32),
            jax.ShapeDtypeStruct((D, B), jnp.float32),
        ],
        scratch_types=[
            pltpu.VMEM((b_per_w,), jnp.int32),
            pltpu.VMEM((b_per_w,), jnp.int32),
            pltpu.VMEM((D, b_per_w), jnp.float32),
            pltpu.VMEM((D, b_per_w), jnp.float32),
            pltpu.SemaphoreType.DMA,
            pltpu.SemaphoreType.DMA,
        ],
    )
    def gather_k(user_t, movie_t, uid, mid, out_u, out_m,
                 uidx, midx, urows, mrows, usem, msem):
        wid = lax.axis_index("s") * NC + lax.axis_index("c")
        base = wid * b_per_w
        pltpu.sync_copy(uid.at[pl.ds(base, b_per_w)], uidx)
        pltpu.sync_copy(mid.at[pl.ds(base, b_per_w)], midx)
        for c0 in range(0, D, DRAIN_C):
            for c in range(c0, c0 + DRAIN_C):
                for j in range(n_chunks):
                    sl = pl.ds(j * IDX_CHUNK, IDX_CHUNK)
                    pltpu.make_async_copy(
                        user_t.at[c].at[uidx.at[sl]],
                        urows.at[c, sl], usem).start()
                    pltpu.make_async_copy(
                        movie_t.at[c].at[midx.at[sl]],
                        mrows.at[c, sl], msem).start()
            pltpu.make_async_copy(
                out_u.at[pl.ds(0, DRAIN_C), pl.ds(0, b_per_w)],
                urows.at[pl.ds(c0, DRAIN_C)], usem).wait()
            pltpu.make_async_copy(
                out_m.at[pl.ds(0, DRAIN_C), pl.ds(0, b_per_w)],
                mrows.at[pl.ds(c0, DRAIN_C)], msem).wait()
        pltpu.sync_copy(urows, out_u.at[:, pl.ds(base, b_per_w)])
        pltpu.sync_copy(mrows, out_m.at[:, pl.ds(base, b_per_w)])

    return gather_k


BLK = 2048


def _dg(w, x):
    return lax.dot_general(w, x, (((0,), (0,)), ((), ())),
                           preferred_element_type=jnp.float32)


def _mlp_body(u_ref, m_ref, c_ref, g_ref, a_ref, o_ref,
              gt_ref, at_ref, ot_ref,
              w0_ref, b0_ref, w1_ref, b1_ref, w2_ref, b2_ref,
              w3_ref, b3_ref, out_ref):
    f32 = jnp.float32
    acc = _dg(w0_ref[0:32, :], u_ref[...])
    acc += _dg(w0_ref[32:64, :], m_ref[...])
    acc += _dg(w0_ref[88:104, :], c_ref[...])

    def small(idx_ref, tab_ref, lo, hi, T):
        oh = (idx_ref[...] ==
              lax.broadcasted_iota(jnp.int32, (T, BLK), 0)).astype(f32)
        return _dg(w0_ref[lo:hi, :], _dg(tab_ref[...], oh))

    acc += small(g_ref, gt_ref, 64, 72, 2)
    acc += small(a_ref, at_ref, 72, 80, 7)
    acc += small(o_ref, ot_ref, 80, 88, 21)
    h = jnp.maximum(acc + b0_ref[...], 0.0)
    h = jnp.maximum(_dg(w1_ref[...], h) + b1_ref[...], 0.0)
    h = jnp.maximum(_dg(w2_ref[...], h) + b2_ref[...], 0.0)
    out_ref[...] = _dg(w3_ref[...], h) + b3_ref[...]


def _full(shape):
    return pl.BlockSpec(shape, lambda i: (0, 0))


def _bcol(rows):
    return pl.BlockSpec((rows, BLK), lambda i: (0, i))


_mlp_call = pl.pallas_call(
    _mlp_body,
    grid=(B // BLK,),
    in_specs=[
        _bcol(D),
        _bcol(D),
        _bcol(16),
        _bcol(1),
        _bcol(1),
        _bcol(1),
        _full((2, 8)), _full((7, 8)), _full((21, 8)),
        _full((104, 128)), _full((128, 1)),
        _full((128, 64)), _full((64, 1)),
        _full((64, 32)), _full((32, 1)),
        _full((32, 1)), _full((1, 1)),
    ],
    out_specs=_bcol(1),
    out_shape=jax.ShapeDtypeStruct((1, B), jnp.float32),
)


def kernel(user_id, movie_id, gender, age, occupation, continuous_features,
           user_table, movie_table, gender_table, age_table, occupation_table,
           W0, b0, W1, b1, W2, b2, W3, b3):
    info = plsc.get_sparse_core_info()
    U, M = _make_gather(info.num_cores, info.num_subcores)(
        user_table.T, movie_table.T,
        user_id.astype(jnp.int32), movie_id.astype(jnp.int32))
    out = _mlp_call(
        U, M, continuous_features.T,
        gender.astype(jnp.int32).reshape(1, B),
        age.astype(jnp.int32).reshape(1, B),
        occupation.astype(jnp.int32).reshape(1, B),
        gender_table, age_table, occupation_table,
        W0, b0.reshape(128, 1), W1, b1.reshape(64, 1),
        W2, b2.reshape(32, 1), W3, b3.reshape(1, 1))
    return out.reshape(B, 1)


# speedup vs baseline: 4.0311x; 4.0311x over previous
import functools

import jax
import jax.numpy as jnp
from jax import lax
from jax.experimental import pallas as pl
from jax.experimental.pallas import tpu as pltpu
from jax.experimental.pallas import tpu_sc as plsc

B = 16384
D = 32
ROWS_PER_GRP = 32
CHUNK = 32


@functools.lru_cache(maxsize=None)
def _make_gather(num_cores, num_subcores):
    NC, NS = num_cores, num_subcores
    NW = NC * NS
    b_per_w = B // NW
    n_chunks = b_per_w // CHUNK
    mesh = plsc.VectorSubcoreMesh(core_axis_name="c", subcore_axis_name="s")

    @functools.partial(
        pl.kernel,
        mesh=mesh,
        compiler_params=pltpu.CompilerParams(use_tc_tiling_on_sc=True),
        out_type=[
            jax.ShapeDtypeStruct((B * D,), jnp.float32),
            jax.ShapeDtypeStruct((B * D,), jnp.float32),
        ],
        scratch_types=[
            pltpu.VMEM((b_per_w,), jnp.int32),
            pltpu.VMEM((b_per_w,), jnp.int32),
            pltpu.VMEM((b_per_w,), jnp.int32),
            pltpu.VMEM((b_per_w,), jnp.int32),
            pltpu.VMEM((CHUNK, 8, 128), jnp.float32),
            pltpu.VMEM((CHUNK, 8, 128), jnp.float32),
            pltpu.VMEM((b_per_w * D,), jnp.float32),
            pltpu.VMEM((b_per_w * D,), jnp.float32),
            pltpu.SemaphoreType.DMA,
            pltpu.SemaphoreType.DMA,
        ],
    )
    def gather_k(user_t, movie_t, uid, mid, out_u, out_m,
                 uidx, midx, ugidx, mgidx, ugrp, mgrp, urows, mrows,
                 usem, msem):
        wid = lax.axis_index("s") * NC + lax.axis_index("c")
        base = wid * b_per_w
        pltpu.sync_copy(uid.at[pl.ds(base, b_per_w)], uidx)
        pltpu.sync_copy(mid.at[pl.ds(base, b_per_w)], midx)
        for v in range(b_per_w // 16):
            sl = pl.ds(v * 16, 16)
            ugidx[sl] = uidx[sl] >> 5
            mgidx[sl] = midx[sl] >> 5

        def chunk_body(k, _):
            o = k * CHUNK
            cu = pltpu.make_async_copy(
                user_t.at[ugidx.at[pl.ds(o, CHUNK)]], ugrp, usem)
            cm = pltpu.make_async_copy(
                movie_t.at[mgidx.at[pl.ds(o, CHUNK)]], mgrp, msem)
            cu.start()
            cm.start()

            def extract(idx_ref, grp, rows):
                for h in range(CHUNK // 16):
                    v = idx_ref[pl.ds(o + h * 16, 16)]
                    for j in range(16):
                        r = v[j]
                        s = (r >> 2) & 7
                        w = (r & 3) * D
                        dst = (o + h * 16 + j) * D
                        rows[pl.ds(dst, 16)] = grp[h * 16 + j, s,
                                                   pl.ds(w, 16)]
                        rows[pl.ds(dst + 16, 16)] = grp[h * 16 + j, s,
                                                        pl.ds(w + 16, 16)]

            cu.wait()
            extract(uidx, ugrp, urows)
            cm.wait()
            extract(midx, mgrp, mrows)
            return 0

        lax.fori_loop(0, n_chunks, chunk_body, 0)
        pltpu.sync_copy(urows, out_u.at[pl.ds(base * D, b_per_w * D)])
        pltpu.sync_copy(mrows, out_m.at[pl.ds(base * D, b_per_w * D)])

    return gather_k


BLK = 2048


def _mlp_body(u_ref, m_ref, c_ref, g_ref, a_ref, o_ref,
              gt_ref, at_ref, ot_ref,
              w0_ref, b0_ref, w1_ref, b1_ref, w2_ref, b2_ref,
              w3_ref, b3_ref, out_ref):
    f32 = jnp.float32
    acc = jnp.dot(u_ref[...], w0_ref[0:32, :], preferred_element_type=f32)
    acc += jnp.dot(m_ref[...], w0_ref[32:64, :], preferred_element_type=f32)
    acc += jnp.dot(c_ref[...], w0_ref[88:104, :], preferred_element_type=f32)

    def small(idx_ref, tab_ref, lo, hi, T):
        oh = (idx_ref[...] ==
              lax.broadcasted_iota(jnp.int32, (BLK, T), 1)).astype(f32)
        e = jnp.dot(oh, tab_ref[...], preferred_element_type=f32)
        return jnp.dot(e, w0_ref[lo:hi, :], preferred_element_type=f32)

    acc += small(g_ref, gt_ref, 64, 72, 2)
    acc += small(a_ref, at_ref, 72, 80, 7)
    acc += small(o_ref, ot_ref, 80, 88, 21)
    h = jnp.maximum(acc + b0_ref[...], 0.0)
    h = jnp.maximum(jnp.dot(h, w1_ref[...], preferred_element_type=f32)
                    + b1_ref[...], 0.0)
    h = jnp.maximum(jnp.dot(h, w2_ref[...], preferred_element_type=f32)
                    + b2_ref[...], 0.0)
    out_ref[...] = (jnp.dot(h, w3_ref[...], preferred_element_type=f32)
                    + b3_ref[...])


def _full(shape):
    return pl.BlockSpec(shape, lambda i: (0, 0))


_mlp_call = pl.pallas_call(
    _mlp_body,
    grid=(B // BLK,),
    in_specs=[
        pl.BlockSpec((BLK, D), lambda i: (i, 0)),
        pl.BlockSpec((BLK, D), lambda i: (i, 0)),
        pl.BlockSpec((BLK, 16), lambda i: (i, 0)),
        pl.BlockSpec((BLK, 1), lambda i: (i, 0)),
        pl.BlockSpec((BLK, 1), lambda i: (i, 0)),
        pl.BlockSpec((BLK, 1), lambda i: (i, 0)),
        _full((2, 8)), _full((7, 8)), _full((21, 8)),
        _full((104, 128)), _full((1, 128)),
        _full((128, 64)), _full((1, 64)),
        _full((64, 32)), _full((1, 32)),
        _full((32, 1)), _full((1, 1)),
    ],
    out_specs=pl.BlockSpec((BLK, 1), lambda i: (i, 0)),
    out_shape=jax.ShapeDtypeStruct((B, 1), jnp.float32),
)


def kernel(user_id, movie_id, gender, age, occupation, continuous_features,
           user_table, movie_table, gender_table, age_table, occupation_table,
           W0, b0, W1, b1, W2, b2, W3, b3):
    info = plsc.get_sparse_core_info()
    Uf, Mf = _make_gather(info.num_cores, info.num_subcores)(
        user_table.reshape(user_table.shape[0] // ROWS_PER_GRP, 8, 128),
        movie_table.reshape(movie_table.shape[0] // ROWS_PER_GRP, 8, 128),
        user_id.astype(jnp.int32), movie_id.astype(jnp.int32))
    return _mlp_call(
        Uf.reshape(B, D), Mf.reshape(B, D), continuous_features,
        gender.astype(jnp.int32).reshape(B, 1),
        age.astype(jnp.int32).reshape(B, 1),
        occupation.astype(jnp.int32).reshape(B, 1),
        gender_table, age_table, occupation_table,
        W0, b0.reshape(1, 128), W1, b1.reshape(1, 64),
        W2, b2.reshape(1, 32), W3, b3.reshape(1, 1))
